# baseline (speedup 1.0000x reference)
import functools

import jax
import jax.numpy as jnp
from jax import lax
from jax.experimental import pallas as pl
from jax.experimental.pallas import tpu as pltpu
from jax.experimental.pallas import tpu_sc as plsc

N_DONORS = 128
N_CLUSTERS = 16
N_VARIANTS = 10000
N_GENES = 20000
N_VXG = 4096
LANES = 16

_NC = 2
_NS = 16
_NW = _NC * _NS
_ROWS_PER_W = N_DONORS // _NW


def _sc_gather_body(genotypes_hbm, sel_hbm, baseline_hbm, vxg_hbm,
                    g_out, base_out,
                    sel_v, vxg_v, row_v0, row_v1, brow_v, out_v0, out_v1,
                    sem_sel, sem_vxg, sem_brow, sem_in0, sem_in1,
                    sem_out0, sem_out1):
    wid = lax.axis_index("s") * _NC + lax.axis_index("c")
    row_bufs = (row_v0, row_v1)
    out_bufs = (out_v0, out_v1)
    in_sems = (sem_in0, sem_in1)
    out_sems = (sem_out0, sem_out1)

    h_sel = pltpu.async_copy(sel_hbm, sel_v, sem_sel)
    h_vxg = pltpu.async_copy(vxg_hbm, vxg_v, sem_vxg)
    brow_src = baseline_hbm.at[lax.min(wid, N_CLUSTERS - 1)]
    h_brow = pltpu.async_copy(brow_src, brow_v, sem_brow)

    def gather_row(idx_v, src_v, dst_v):
        @plsc.parallel_loop(0, N_VXG // LANES, 1, unroll=8)
        def _(j):
            idx = idx_v[pl.ds(j * LANES, LANES)]
            dst_v[pl.ds(j * LANES, LANES)] = plsc.load_gather(src_v, [idx])

    h_in = [pltpu.async_copy(genotypes_hbm.at[wid * _ROWS_PER_W], row_v0,
                             sem_in0)]
    h_out = []
    h_sel.wait()
    for r in range(_ROWS_PER_W):
        b = r % 2
        h_in[r].wait()
        if r + 1 < _ROWS_PER_W:
            h_in.append(pltpu.async_copy(
                genotypes_hbm.at[wid * _ROWS_PER_W + r + 1],
                row_bufs[(r + 1) % 2], in_sems[(r + 1) % 2]))
        if r >= 2:
            h_out[r - 2].wait()
        gather_row(sel_v, row_bufs[b], out_bufs[b])
        h_out.append(pltpu.async_copy(out_bufs[b],
                                      g_out.at[wid * _ROWS_PER_W + r],
                                      out_sems[b]))

    h_vxg.wait()
    h_brow.wait()
    h_out[_ROWS_PER_W - 2].wait()

    @pl.when(wid < N_CLUSTERS)
    def _():
        gather_row(vxg_v, brow_v, out_bufs[0])
        pltpu.sync_copy(out_bufs[0], base_out.at[wid])

    h_out[_ROWS_PER_W - 1].wait()


_sc_gather = functools.partial(
    pl.kernel,
    out_type=[
        jax.ShapeDtypeStruct((N_DONORS, N_VXG), jnp.float32),
        jax.ShapeDtypeStruct((N_CLUSTERS, N_VXG), jnp.float32),
    ],
    mesh=plsc.VectorSubcoreMesh(core_axis_name="c", subcore_axis_name="s"),
    scratch_types=[
        pltpu.VMEM((N_VXG,), jnp.int32),
        pltpu.VMEM((N_VXG,), jnp.int32),
        pltpu.VMEM((N_VARIANTS,), jnp.float32),
        pltpu.VMEM((N_VARIANTS,), jnp.float32),
        pltpu.VMEM((N_GENES,), jnp.float32),
        pltpu.VMEM((N_VXG,), jnp.float32),
        pltpu.VMEM((N_VXG,), jnp.float32),
        pltpu.SemaphoreType.DMA,
        pltpu.SemaphoreType.DMA,
        pltpu.SemaphoreType.DMA,
        pltpu.SemaphoreType.DMA,
        pltpu.SemaphoreType.DMA,
        pltpu.SemaphoreType.DMA,
        pltpu.SemaphoreType.DMA,
    ],
    compiler_params=pltpu.CompilerParams(needs_layout_passes=False),
)(_sc_gather_body)


_D_BLK = 8
_V_BLK = 4096


_K_DMA = 8
_D_CHK = _D_BLK // _K_DMA


def _tc_dense_body(g_ref, base_ref, fc_ref, lib_ref, out_hbm,
                   buf0, buf1, sem0, sem1):
    i = pl.program_id(0)
    nsteps = pl.num_programs(0)
    b = base_ref[...]
    f = fc_ref[...]

    def compute(buf):
        for d in range(_D_BLK):
            gd = g_ref[d, :][None, :]
            ld = lib_ref[d, :][:, None]
            buf[d, :, :] = jnp.exp(b + gd * f) * ld

    def fire(buf, sem, step):
        for k in range(_K_DMA):
            pltpu.make_async_copy(
                buf.at[pl.ds(k * _D_CHK, _D_CHK)],
                out_hbm.at[pl.ds(step * _D_BLK + k * _D_CHK, _D_CHK)],
                sem.at[k]).start()

    def drain(buf, sem, step):
        for k in range(_K_DMA):
            pltpu.make_async_copy(
                buf.at[pl.ds(k * _D_CHK, _D_CHK)],
                out_hbm.at[pl.ds(step * _D_BLK + k * _D_CHK, _D_CHK)],
                sem.at[k]).wait()

    even = i % 2 == 0

    @pl.when(jnp.logical_and(even, i >= 2))
    def _():
        drain(buf0, sem0, i - 2)

    @pl.when(jnp.logical_and(jnp.logical_not(even), i >= 2))
    def _():
        drain(buf1, sem1, i - 2)

    @pl.when(even)
    def _():
        compute(buf0)
        fire(buf0, sem0, i)

    @pl.when(jnp.logical_not(even))
    def _():
        compute(buf1)
        fire(buf1, sem1, i)

    @pl.when(i == nsteps - 1)
    def _():
        drain(buf0, sem0, i - 1)
        drain(buf1, sem1, i)


def kernel(fc_log, genotypes, expression_obs, variantxgene_to_gene,
           local_variant_to_local_variantxgene_selector, variantxgene_to_local_gene,
           lib, baseline_log, dispersion_log):
    del expression_obs, variantxgene_to_local_gene, dispersion_log
    g, base = _sc_gather(genotypes, local_variant_to_local_variantxgene_selector,
                         baseline_log, variantxgene_to_gene)
    out = pl.pallas_call(
        _tc_dense_body,
        grid=(N_DONORS // _D_BLK,),
        in_specs=[
            pl.BlockSpec((_D_BLK, N_VXG), lambda i: (i, 0)),
            pl.BlockSpec((N_CLUSTERS, N_VXG), lambda i: (0, 0)),
            pl.BlockSpec((N_CLUSTERS, N_VXG), lambda i: (0, 0)),
            pl.BlockSpec((_D_BLK, N_CLUSTERS), lambda i: (i, 0)),
        ],
        out_specs=pl.BlockSpec(memory_space=pl.ANY),
        out_shape=jax.ShapeDtypeStruct((N_DONORS, N_CLUSTERS, N_VXG), jnp.float32),
        scratch_shapes=[
            pltpu.VMEM((_D_BLK, N_CLUSTERS, N_VXG), jnp.float32),
            pltpu.VMEM((_D_BLK, N_CLUSTERS, N_VXG), jnp.float32),
            pltpu.SemaphoreType.DMA((_K_DMA,)),
            pltpu.SemaphoreType.DMA((_K_DMA,)),
        ],
        compiler_params=pltpu.CompilerParams(
            dimension_semantics=("arbitrary",)),
    )(g, base, fc_log, lib)
    return out


# speedup vs baseline: 1.0086x; 1.0086x over previous
import functools

import jax
import jax.numpy as jnp
from jax import lax
from jax.experimental import pallas as pl
from jax.experimental.pallas import tpu as pltpu
from jax.experimental.pallas import tpu_sc as plsc

N_DONORS = 128
N_CLUSTERS = 16
N_VARIANTS = 10000
N_GENES = 20000
N_VXG = 4096
LANES = 16

_LOG2E = 1.4426950408889634

_NC = 2
_NS = 16
_NW = _NC * _NS
_ROWS_PER_W = N_DONORS // _NW


def _sc_gather_body(genotypes_hbm, sel_hbm, baseline_hbm, vxg_hbm,
                    g_out, base_out,
                    sel_v, vxg_v, row_v0, row_v1, brow_v, out_v0, out_v1,
                    sem_sel, sem_vxg, sem_brow, sem_in0, sem_in1,
                    sem_out0, sem_out1):
    wid = lax.axis_index("s") * _NC + lax.axis_index("c")
    row_bufs = (row_v0, row_v1)
    out_bufs = (out_v0, out_v1)
    in_sems = (sem_in0, sem_in1)
    out_sems = (sem_out0, sem_out1)

    h_sel = pltpu.async_copy(sel_hbm, sel_v, sem_sel)
    h_vxg = pltpu.async_copy(vxg_hbm, vxg_v, sem_vxg)
    brow_src = baseline_hbm.at[lax.min(wid, N_CLUSTERS - 1)]
    h_brow = pltpu.async_copy(brow_src, brow_v, sem_brow)

    def gather_row(idx_v, src_v, dst_v, scale=None):
        @plsc.parallel_loop(0, N_VXG // LANES, 1, unroll=8)
        def _(j):
            idx = idx_v[pl.ds(j * LANES, LANES)]
            vals = plsc.load_gather(src_v, [idx])
            if scale is not None:
                vals = vals * scale
            dst_v[pl.ds(j * LANES, LANES)] = vals

    h_in = [pltpu.async_copy(genotypes_hbm.at[wid * _ROWS_PER_W], row_v0,
                             sem_in0)]
    h_out = []
    h_sel.wait()
    for r in range(_ROWS_PER_W):
        b = r % 2
        h_in[r].wait()
        if r + 1 < _ROWS_PER_W:
            h_in.append(pltpu.async_copy(
                genotypes_hbm.at[wid * _ROWS_PER_W + r + 1],
                row_bufs[(r + 1) % 2], in_sems[(r + 1) % 2]))
        if r >= 2:
            h_out[r - 2].wait()
        gather_row(sel_v, row_bufs[b], out_bufs[b])
        h_out.append(pltpu.async_copy(out_bufs[b],
                                      g_out.at[wid * _ROWS_PER_W + r],
                                      out_sems[b]))

    h_vxg.wait()
    h_brow.wait()
    h_out[_ROWS_PER_W - 2].wait()

    @pl.when(wid < N_CLUSTERS)
    def _():
        gather_row(vxg_v, brow_v, out_bufs[0], scale=jnp.float32(_LOG2E))
        pltpu.sync_copy(out_bufs[0], base_out.at[wid])

    h_out[_ROWS_PER_W - 1].wait()


_sc_gather = functools.partial(
    pl.kernel,
    out_type=[
        jax.ShapeDtypeStruct((N_DONORS, N_VXG), jnp.float32),
        jax.ShapeDtypeStruct((N_CLUSTERS, N_VXG), jnp.float32),
    ],
    mesh=plsc.VectorSubcoreMesh(core_axis_name="c", subcore_axis_name="s"),
    scratch_types=[
        pltpu.VMEM((N_VXG,), jnp.int32),
        pltpu.VMEM((N_VXG,), jnp.int32),
        pltpu.VMEM((N_VARIANTS,), jnp.float32),
        pltpu.VMEM((N_VARIANTS,), jnp.float32),
        pltpu.VMEM((N_GENES,), jnp.float32),
        pltpu.VMEM((N_VXG,), jnp.float32),
        pltpu.VMEM((N_VXG,), jnp.float32),
        pltpu.SemaphoreType.DMA,
        pltpu.SemaphoreType.DMA,
        pltpu.SemaphoreType.DMA,
        pltpu.SemaphoreType.DMA,
        pltpu.SemaphoreType.DMA,
        pltpu.SemaphoreType.DMA,
        pltpu.SemaphoreType.DMA,
    ],
    compiler_params=pltpu.CompilerParams(needs_layout_passes=False),
)(_sc_gather_body)


_D_BLK = 8
_V_BLK = 4096


_K_DMA = 8
_D_CHK = _D_BLK // _K_DMA


def _tc_dense_body(g_ref, base_ref, fc_ref, lib_ref, out_hbm,
                   buf0, buf1, sem0, sem1):
    i = pl.program_id(0)
    nsteps = pl.num_programs(0)
    b = base_ref[...]
    f = fc_ref[...]

    def compute(buf):
        for d in range(_D_BLK):
            gd = g_ref[pl.ds(d, 1), :]
            ld = lib_ref[d, :][:, None]
            buf[d, :, :] = jnp.exp2(b + gd * f) * ld

    def fire(buf, sem, step):
        for k in range(_K_DMA):
            pltpu.make_async_copy(
                buf.at[pl.ds(k * _D_CHK, _D_CHK)],
                out_hbm.at[pl.ds(step * _D_BLK + k * _D_CHK, _D_CHK)],
                sem.at[k]).start()

    def drain(buf, sem, step):
        for k in range(_K_DMA):
            pltpu.make_async_copy(
                buf.at[pl.ds(k * _D_CHK, _D_CHK)],
                out_hbm.at[pl.ds(step * _D_BLK + k * _D_CHK, _D_CHK)],
                sem.at[k]).wait()

    even = i % 2 == 0

    @pl.when(jnp.logical_and(even, i >= 2))
    def _():
        drain(buf0, sem0, i - 2)

    @pl.when(jnp.logical_and(jnp.logical_not(even), i >= 2))
    def _():
        drain(buf1, sem1, i - 2)

    @pl.when(even)
    def _():
        compute(buf0)
        fire(buf0, sem0, i)

    @pl.when(jnp.logical_not(even))
    def _():
        compute(buf1)
        fire(buf1, sem1, i)

    @pl.when(i == nsteps - 1)
    def _():
        drain(buf0, sem0, i - 1)
        drain(buf1, sem1, i)


def kernel(fc_log, genotypes, expression_obs, variantxgene_to_gene,
           local_variant_to_local_variantxgene_selector, variantxgene_to_local_gene,
           lib, baseline_log, dispersion_log):
    del expression_obs, variantxgene_to_local_gene, dispersion_log
    g, base2 = _sc_gather(genotypes, local_variant_to_local_variantxgene_selector,
                          baseline_log, variantxgene_to_gene)
    fc2 = fc_log * jnp.float32(_LOG2E)
    out = pl.pallas_call(
        _tc_dense_body,
        grid=(N_DONORS // _D_BLK,),
        in_specs=[
            pl.BlockSpec((_D_BLK, N_VXG), lambda i: (i, 0)),
            pl.BlockSpec((N_CLUSTERS, N_VXG), lambda i: (0, 0)),
            pl.BlockSpec((N_CLUSTERS, N_VXG), lambda i: (0, 0)),
            pl.BlockSpec((_D_BLK, N_CLUSTERS), lambda i: (i, 0)),
        ],
        out_specs=pl.BlockSpec(memory_space=pl.ANY),
        out_shape=jax.ShapeDtypeStruct((N_DONORS, N_CLUSTERS, N_VXG), jnp.float32),
        scratch_shapes=[
            pltpu.VMEM((_D_BLK, N_CLUSTERS, N_VXG), jnp.float32),
            pltpu.VMEM((_D_BLK, N_CLUSTERS, N_VXG), jnp.float32),
            pltpu.SemaphoreType.DMA((_K_DMA,)),
            pltpu.SemaphoreType.DMA((_K_DMA,)),
        ],
        compiler_params=pltpu.CompilerParams(
            dimension_semantics=("arbitrary",)),
    )(g, base2, fc2, lib)
    return out


# speedup vs baseline: 1.0183x; 1.0096x over previous
import functools

import jax
import jax.numpy as jnp
from jax import lax
from jax.experimental import pallas as pl
from jax.experimental.pallas import tpu as pltpu
from jax.experimental.pallas import tpu_sc as plsc

N_DONORS = 128
N_CLUSTERS = 16
N_VARIANTS = 10000
N_GENES = 20000
N_VXG = 4096
LANES = 16

_LOG2E = 1.4426950408889634

_NC = 2
_NS = 16
_NW = _NC * _NS
_ROWS_PER_W = N_DONORS // _NW


def _sc_gather_body(genotypes_hbm, sel_hbm, baseline_hbm, vxg_hbm,
                    g_out, base_out,
                    sel_v, vxg_v, row_v0, row_v1, brow_v, out_v0, out_v1,
                    sem_sel, sem_vxg, sem_brow, sem_in0, sem_in1,
                    sem_out0, sem_out1):
    wid = lax.axis_index("s") * _NC + lax.axis_index("c")
    row_bufs = (row_v0, row_v1)
    out_bufs = (out_v0, out_v1)
    in_sems = (sem_in0, sem_in1)
    out_sems = (sem_out0, sem_out1)

    h_sel = pltpu.async_copy(sel_hbm, sel_v, sem_sel)
    h_vxg = pltpu.async_copy(vxg_hbm, vxg_v, sem_vxg)
    brow_src = baseline_hbm.at[lax.min(wid, N_CLUSTERS - 1)]
    h_brow = pltpu.async_copy(brow_src, brow_v, sem_brow)

    def gather_row(idx_v, src_v, dst_v, scale=None):
        @plsc.parallel_loop(0, N_VXG // LANES, 1, unroll=8)
        def _(j):
            idx = idx_v[pl.ds(j * LANES, LANES)]
            vals = plsc.load_gather(src_v, [idx])
            if scale is not None:
                vals = vals * scale
            dst_v[pl.ds(j * LANES, LANES)] = vals

    h_in = [pltpu.async_copy(genotypes_hbm.at[wid * _ROWS_PER_W], row_v0,
                             sem_in0)]
    h_out = []
    h_sel.wait()
    for r in range(_ROWS_PER_W):
        b = r % 2
        h_in[r].wait()
        if r + 1 < _ROWS_PER_W:
            h_in.append(pltpu.async_copy(
                genotypes_hbm.at[wid * _ROWS_PER_W + r + 1],
                row_bufs[(r + 1) % 2], in_sems[(r + 1) % 2]))
        if r >= 2:
            h_out[r - 2].wait()
        gather_row(sel_v, row_bufs[b], out_bufs[b])
        h_out.append(pltpu.async_copy(out_bufs[b],
                                      g_out.at[wid * _ROWS_PER_W + r],
                                      out_sems[b]))

    h_vxg.wait()
    h_brow.wait()
    h_out[_ROWS_PER_W - 2].wait()

    @pl.when(wid < N_CLUSTERS)
    def _():
        gather_row(vxg_v, brow_v, out_bufs[0], scale=jnp.float32(_LOG2E))
        pltpu.sync_copy(out_bufs[0], base_out.at[wid])

    h_out[_ROWS_PER_W - 1].wait()


_sc_gather = functools.partial(
    pl.kernel,
    out_type=[
        jax.ShapeDtypeStruct((N_DONORS, N_VXG), jnp.float32),
        jax.ShapeDtypeStruct((N_CLUSTERS, N_VXG), jnp.float32),
    ],
    mesh=plsc.VectorSubcoreMesh(core_axis_name="c", subcore_axis_name="s"),
    scratch_types=[
        pltpu.VMEM((N_VXG,), jnp.int32),
        pltpu.VMEM((N_VXG,), jnp.int32),
        pltpu.VMEM((N_VARIANTS,), jnp.float32),
        pltpu.VMEM((N_VARIANTS,), jnp.float32),
        pltpu.VMEM((N_GENES,), jnp.float32),
        pltpu.VMEM((N_VXG,), jnp.float32),
        pltpu.VMEM((N_VXG,), jnp.float32),
        pltpu.SemaphoreType.DMA,
        pltpu.SemaphoreType.DMA,
        pltpu.SemaphoreType.DMA,
        pltpu.SemaphoreType.DMA,
        pltpu.SemaphoreType.DMA,
        pltpu.SemaphoreType.DMA,
        pltpu.SemaphoreType.DMA,
    ],
    compiler_params=pltpu.CompilerParams(needs_layout_passes=False,
                                         use_tc_tiling_on_sc=True),
)(_sc_gather_body)


_D_BLK = 8
_V_BLK = 4096


_K_DMA = 8
_D_CHK = _D_BLK // _K_DMA


def _tc_dense_body(g_ref, base_ref, fc_ref, lib_ref, out_hbm,
                   buf0, buf1, sem0, sem1):
    i = pl.program_id(0)
    nsteps = pl.num_programs(0)
    b = base_ref[...]
    f = fc_ref[...]

    def compute(buf):
        for d in range(_D_BLK):
            gd = g_ref[pl.ds(d, 1), :]
            ld = lib_ref[d, :][:, None]
            buf[d, :, :] = jnp.exp2(b + gd * f) * ld

    def fire(buf, sem, step):
        for k in range(_K_DMA):
            pltpu.make_async_copy(
                buf.at[pl.ds(k * _D_CHK, _D_CHK)],
                out_hbm.at[pl.ds(step * _D_BLK + k * _D_CHK, _D_CHK)],
                sem.at[k]).start()

    def drain(buf, sem, step):
        for k in range(_K_DMA):
            pltpu.make_async_copy(
                buf.at[pl.ds(k * _D_CHK, _D_CHK)],
                out_hbm.at[pl.ds(step * _D_BLK + k * _D_CHK, _D_CHK)],
                sem.at[k]).wait()

    even = i % 2 == 0

    @pl.when(jnp.logical_and(even, i >= 2))
    def _():
        drain(buf0, sem0, i - 2)

    @pl.when(jnp.logical_and(jnp.logical_not(even), i >= 2))
    def _():
        drain(buf1, sem1, i - 2)

    @pl.when(even)
    def _():
        compute(buf0)
        fire(buf0, sem0, i)

    @pl.when(jnp.logical_not(even))
    def _():
        compute(buf1)
        fire(buf1, sem1, i)

    @pl.when(i == nsteps - 1)
    def _():
        drain(buf0, sem0, i - 1)
        drain(buf1, sem1, i)


def kernel(fc_log, genotypes, expression_obs, variantxgene_to_gene,
           local_variant_to_local_variantxgene_selector, variantxgene_to_local_gene,
           lib, baseline_log, dispersion_log):
    del expression_obs, variantxgene_to_local_gene, dispersion_log
    g, base2 = _sc_gather(genotypes, local_variant_to_local_variantxgene_selector,
                          baseline_log, variantxgene_to_gene)
    fc2 = fc_log * jnp.float32(_LOG2E)
    out = pl.pallas_call(
        _tc_dense_body,
        grid=(N_DONORS // _D_BLK,),
        in_specs=[
            pl.BlockSpec((_D_BLK, N_VXG), lambda i: (i, 0)),
            pl.BlockSpec((N_CLUSTERS, N_VXG), lambda i: (0, 0)),
            pl.BlockSpec((N_CLUSTERS, N_VXG), lambda i: (0, 0)),
            pl.BlockSpec((_D_BLK, N_CLUSTERS), lambda i: (i, 0)),
        ],
        out_specs=pl.BlockSpec(memory_space=pl.ANY),
        out_shape=jax.ShapeDtypeStruct((N_DONORS, N_CLUSTERS, N_VXG), jnp.float32),
        scratch_shapes=[
            pltpu.VMEM((_D_BLK, N_CLUSTERS, N_VXG), jnp.float32),
            pltpu.VMEM((_D_BLK, N_CLUSTERS, N_VXG), jnp.float32),
            pltpu.SemaphoreType.DMA((_K_DMA,)),
            pltpu.SemaphoreType.DMA((_K_DMA,)),
        ],
        compiler_params=pltpu.CompilerParams(
            dimension_semantics=("arbitrary",)),
    )(g, base2, fc2, lib)
    return out


# speedup vs baseline: 1.1087x; 1.0889x over previous
import functools

import jax
import jax.numpy as jnp
from jax import lax
from jax.experimental import pallas as pl
from jax.experimental.pallas import tpu as pltpu
from jax.experimental.pallas import tpu_sc as plsc

N_DONORS = 128
N_CLUSTERS = 16
N_VARIANTS = 10000
N_GENES = 20000
N_VXG = 4096
LANES = 16

_LOG2E = 1.4426950408889634

_NC = 2
_NS = 16
_NW = _NC * _NS
_ROWS_PER_W = N_DONORS // _NW


_J_PER_W = N_VXG // _NW


def _sc_gather_body(genoT_hbm, sel_hbm, baseline_hbm, vxg_hbm,
                    gT_out, base_out,
                    selc_v, vxg_v, rows_v, brow_v, bout_v,
                    sem_selc, sem_vxg, sem_brow, sem_rows):
    wid = lax.axis_index("s") * _NC + lax.axis_index("c")
    j0 = wid * _J_PER_W

    h_selc = pltpu.async_copy(sel_hbm.at[pl.ds(j0, _J_PER_W)], selc_v,
                              sem_selc)
    h_vxg = pltpu.async_copy(vxg_hbm, vxg_v, sem_vxg)
    brow_src = baseline_hbm.at[lax.min(wid, N_CLUSTERS - 1)]
    h_brow = pltpu.async_copy(brow_src, brow_v, sem_brow)

    h_selc.wait()
    pltpu.async_copy(genoT_hbm.at[selc_v], rows_v, sem_rows).wait()
    pltpu.sync_copy(rows_v, gT_out.at[pl.ds(j0, _J_PER_W)])

    h_vxg.wait()
    h_brow.wait()

    @pl.when(wid < N_CLUSTERS)
    def _():
        @plsc.parallel_loop(0, N_VXG // LANES, 1, unroll=8)
        def _(j):
            idx = vxg_v[pl.ds(j * LANES, LANES)]
            vals = plsc.load_gather(brow_v, [idx]) * jnp.float32(_LOG2E)
            bout_v[pl.ds(j * LANES, LANES)] = vals

        pltpu.sync_copy(bout_v, base_out.at[wid])


_sc_gather = functools.partial(
    pl.kernel,
    out_type=[
        jax.ShapeDtypeStruct((N_VXG, N_DONORS), jnp.float32),
        jax.ShapeDtypeStruct((N_CLUSTERS, N_VXG), jnp.float32),
    ],
    mesh=plsc.VectorSubcoreMesh(core_axis_name="c", subcore_axis_name="s"),
    scratch_types=[
        pltpu.VMEM((_J_PER_W,), jnp.int32),
        pltpu.VMEM((N_VXG,), jnp.int32),
        pltpu.VMEM((_J_PER_W, N_DONORS), jnp.float32),
        pltpu.VMEM((N_GENES,), jnp.float32),
        pltpu.VMEM((N_VXG,), jnp.float32),
        pltpu.SemaphoreType.DMA,
        pltpu.SemaphoreType.DMA,
        pltpu.SemaphoreType.DMA,
        pltpu.SemaphoreType.DMA,
    ],
    compiler_params=pltpu.CompilerParams(needs_layout_passes=False,
                                         use_tc_tiling_on_sc=True),
)(_sc_gather_body)


_D_BLK = 8
_V_BLK = 4096


_K_DMA = 8
_D_CHK = _D_BLK // _K_DMA


def _tc_dense_body(g_ref, base_ref, fc_ref, lib_ref, out_hbm,
                   buf0, buf1, sem0, sem1):
    i = pl.program_id(0)
    nsteps = pl.num_programs(0)
    b = base_ref[...]
    f = fc_ref[...]

    def compute(buf):
        for d in range(_D_BLK):
            gd = g_ref[pl.ds(d, 1), :]
            ld = lib_ref[d, :][:, None]
            buf[d, :, :] = jnp.exp2(b + gd * f) * ld

    def fire(buf, sem, step):
        for k in range(_K_DMA):
            pltpu.make_async_copy(
                buf.at[pl.ds(k * _D_CHK, _D_CHK)],
                out_hbm.at[pl.ds(step * _D_BLK + k * _D_CHK, _D_CHK)],
                sem.at[k]).start()

    def drain(buf, sem, step):
        for k in range(_K_DMA):
            pltpu.make_async_copy(
                buf.at[pl.ds(k * _D_CHK, _D_CHK)],
                out_hbm.at[pl.ds(step * _D_BLK + k * _D_CHK, _D_CHK)],
                sem.at[k]).wait()

    even = i % 2 == 0

    @pl.when(jnp.logical_and(even, i >= 2))
    def _():
        drain(buf0, sem0, i - 2)

    @pl.when(jnp.logical_and(jnp.logical_not(even), i >= 2))
    def _():
        drain(buf1, sem1, i - 2)

    @pl.when(even)
    def _():
        compute(buf0)
        fire(buf0, sem0, i)

    @pl.when(jnp.logical_not(even))
    def _():
        compute(buf1)
        fire(buf1, sem1, i)

    @pl.when(i == nsteps - 1)
    def _():
        drain(buf0, sem0, i - 1)
        drain(buf1, sem1, i)


def kernel(fc_log, genotypes, expression_obs, variantxgene_to_gene,
           local_variant_to_local_variantxgene_selector, variantxgene_to_local_gene,
           lib, baseline_log, dispersion_log):
    del expression_obs, variantxgene_to_local_gene, dispersion_log
    genoT = jnp.swapaxes(genotypes, 0, 1)
    gT, base2 = _sc_gather(genoT, local_variant_to_local_variantxgene_selector,
                           baseline_log, variantxgene_to_gene)
    g = jnp.swapaxes(gT, 0, 1)
    fc2 = fc_log * jnp.float32(_LOG2E)
    out = pl.pallas_call(
        _tc_dense_body,
        grid=(N_DONORS // _D_BLK,),
        in_specs=[
            pl.BlockSpec((_D_BLK, N_VXG), lambda i: (i, 0)),
            pl.BlockSpec((N_CLUSTERS, N_VXG), lambda i: (0, 0)),
            pl.BlockSpec((N_CLUSTERS, N_VXG), lambda i: (0, 0)),
            pl.BlockSpec((_D_BLK, N_CLUSTERS), lambda i: (i, 0)),
        ],
        out_specs=pl.BlockSpec(memory_space=pl.ANY),
        out_shape=jax.ShapeDtypeStruct((N_DONORS, N_CLUSTERS, N_VXG), jnp.float32),
        scratch_shapes=[
            pltpu.VMEM((_D_BLK, N_CLUSTERS, N_VXG), jnp.float32),
            pltpu.VMEM((_D_BLK, N_CLUSTERS, N_VXG), jnp.float32),
            pltpu.SemaphoreType.DMA((_K_DMA,)),
            pltpu.SemaphoreType.DMA((_K_DMA,)),
        ],
        compiler_params=pltpu.CompilerParams(
            dimension_semantics=("arbitrary",)),
    )(g, base2, fc2, lib)
    return out


# speedup vs baseline: 1.2034x; 1.0854x over previous
import functools

import jax
import jax.numpy as jnp
from jax import lax
from jax.experimental import pallas as pl
from jax.experimental.pallas import tpu as pltpu
from jax.experimental.pallas import tpu_sc as plsc

N_DONORS = 128
N_CLUSTERS = 16
N_VARIANTS = 10000
N_GENES = 20000
N_VXG = 4096
LANES = 16

_LOG2E = 1.4426950408889634

_NC = 2
_NS = 16
_NW = _NC * _NS
_ROWS_PER_W = N_DONORS // _NW


_J_PER_W = N_VXG // _NW


def _sc_gather_body(genoT_hbm, sel_hbm, baseline_hbm, vxg_hbm,
                    gT_out, base_out,
                    selc_v, vxg_v, rows_v, brow_v, bout_v,
                    sem_selc, sem_vxg, sem_brow, sem_rows):
    wid = lax.axis_index("s") * _NC + lax.axis_index("c")
    j0 = wid * _J_PER_W

    h_selc = pltpu.async_copy(sel_hbm.at[pl.ds(j0, _J_PER_W)], selc_v,
                              sem_selc)
    h_vxg = pltpu.async_copy(vxg_hbm, vxg_v, sem_vxg)
    brow_src = baseline_hbm.at[lax.min(wid, N_CLUSTERS - 1)]
    h_brow = pltpu.async_copy(brow_src, brow_v, sem_brow)

    h_selc.wait()
    pltpu.async_copy(genoT_hbm.at[selc_v], rows_v, sem_rows).wait()
    pltpu.sync_copy(rows_v, gT_out.at[pl.ds(j0, _J_PER_W)])

    h_vxg.wait()
    h_brow.wait()

    @pl.when(wid < N_CLUSTERS)
    def _():
        @plsc.parallel_loop(0, N_VXG // LANES, 1, unroll=8)
        def _(j):
            idx = vxg_v[pl.ds(j * LANES, LANES)]
            vals = plsc.load_gather(brow_v, [idx]) * jnp.float32(_LOG2E)
            bout_v[pl.ds(j * LANES, LANES)] = vals

        pltpu.sync_copy(bout_v, base_out.at[wid])


_sc_gather = functools.partial(
    pl.kernel,
    out_type=[
        jax.ShapeDtypeStruct((N_VXG, N_DONORS), jnp.float32),
        jax.ShapeDtypeStruct((N_CLUSTERS, N_VXG), jnp.float32),
    ],
    mesh=plsc.VectorSubcoreMesh(core_axis_name="c", subcore_axis_name="s"),
    scratch_types=[
        pltpu.VMEM((_J_PER_W,), jnp.int32),
        pltpu.VMEM((N_VXG,), jnp.int32),
        pltpu.VMEM((_J_PER_W, N_DONORS), jnp.float32),
        pltpu.VMEM((N_GENES,), jnp.float32),
        pltpu.VMEM((N_VXG,), jnp.float32),
        pltpu.SemaphoreType.DMA,
        pltpu.SemaphoreType.DMA,
        pltpu.SemaphoreType.DMA,
        pltpu.SemaphoreType.DMA,
    ],
    compiler_params=pltpu.CompilerParams(needs_layout_passes=False,
                                         use_tc_tiling_on_sc=True),
)(_sc_gather_body)


_D_BLK = 8
_V_BLK = 4096


_K_DMA = 8
_D_CHK = _D_BLK // _K_DMA


_V_CHUNK = 512


def _tc_dense_body(gT_ref, base_ref, fc_ref, lib_ref, out_hbm,
                   gvm, buf0, buf1, sem0, sem1):
    i = pl.program_id(0)
    nsteps = pl.num_programs(0)

    @pl.when(i == 0)
    def _():
        gvm[...] = gT_ref[...].T

    def compute(buf):
        for v0 in range(0, N_VXG, _V_CHUNK):
            vs = pl.ds(v0, _V_CHUNK)
            bq = base_ref[:, vs]
            fq = fc_ref[:, vs]
            for d in range(_D_BLK):
                gd = gvm[pl.ds(i * _D_BLK + d, 1), vs]
                ld = lib_ref[d, :][:, None]
                buf[d, :, vs] = jnp.exp2(bq + gd * fq) * ld

    def fire(buf, sem, step):
        for k in range(_K_DMA):
            pltpu.make_async_copy(
                buf.at[pl.ds(k * _D_CHK, _D_CHK)],
                out_hbm.at[pl.ds(step * _D_BLK + k * _D_CHK, _D_CHK)],
                sem.at[k]).start()

    def drain(buf, sem, step):
        for k in range(_K_DMA):
            pltpu.make_async_copy(
                buf.at[pl.ds(k * _D_CHK, _D_CHK)],
                out_hbm.at[pl.ds(step * _D_BLK + k * _D_CHK, _D_CHK)],
                sem.at[k]).wait()

    even = i % 2 == 0

    @pl.when(jnp.logical_and(even, i >= 2))
    def _():
        drain(buf0, sem0, i - 2)

    @pl.when(jnp.logical_and(jnp.logical_not(even), i >= 2))
    def _():
        drain(buf1, sem1, i - 2)

    @pl.when(even)
    def _():
        compute(buf0)
        fire(buf0, sem0, i)

    @pl.when(jnp.logical_not(even))
    def _():
        compute(buf1)
        fire(buf1, sem1, i)

    @pl.when(i == nsteps - 1)
    def _():
        drain(buf0, sem0, i - 1)
        drain(buf1, sem1, i)


def kernel(fc_log, genotypes, expression_obs, variantxgene_to_gene,
           local_variant_to_local_variantxgene_selector, variantxgene_to_local_gene,
           lib, baseline_log, dispersion_log):
    del expression_obs, variantxgene_to_local_gene, dispersion_log
    genoT = jnp.swapaxes(genotypes, 0, 1)
    gT, base2 = _sc_gather(genoT, local_variant_to_local_variantxgene_selector,
                           baseline_log, variantxgene_to_gene)
    fc2 = fc_log * jnp.float32(_LOG2E)
    out = pl.pallas_call(
        _tc_dense_body,
        grid=(N_DONORS // _D_BLK,),
        in_specs=[
            pl.BlockSpec((N_VXG, N_DONORS), lambda i: (0, 0)),
            pl.BlockSpec((N_CLUSTERS, N_VXG), lambda i: (0, 0)),
            pl.BlockSpec((N_CLUSTERS, N_VXG), lambda i: (0, 0)),
            pl.BlockSpec((_D_BLK, N_CLUSTERS), lambda i: (i, 0)),
        ],
        out_specs=pl.BlockSpec(memory_space=pl.ANY),
        out_shape=jax.ShapeDtypeStruct((N_DONORS, N_CLUSTERS, N_VXG), jnp.float32),
        scratch_shapes=[
            pltpu.VMEM((N_DONORS, N_VXG), jnp.float32),
            pltpu.VMEM((_D_BLK, N_CLUSTERS, N_VXG), jnp.float32),
            pltpu.VMEM((_D_BLK, N_CLUSTERS, N_VXG), jnp.float32),
            pltpu.SemaphoreType.DMA((_K_DMA,)),
            pltpu.SemaphoreType.DMA((_K_DMA,)),
        ],
        compiler_params=pltpu.CompilerParams(
            dimension_semantics=("arbitrary",)),
    )(gT, base2, fc2, lib)
    return out
